# baseline (speedup 1.0000x reference)
import jax
import jax.numpy as jnp
from jax.experimental import pallas as pl
from jax.experimental.pallas import tpu as pltpu

N_NODES = 10000
D_FEAT = 128
K = 32
EMB = 128
N_SEEDS = 2500
E = N_SEEDS * K

ROWS = 80
COLS = 128
N_PAD = ROWS * COLS

SPB = 100
EPB = SPB * K
NBLK = N_SEEDS // SPB

MOM_BLK = 8000
NMOM = E // MOM_BLK

EPS = 1e-5


def _fps_body(px_ref, py_ref, pz_ref, idx_ref):
    px = px_ref[...]
    py = py_ref[...]
    pz = pz_ref[...]
    row = jax.lax.broadcasted_iota(jnp.int32, (ROWS, COLS), 0)
    col = jax.lax.broadcasted_iota(jnp.int32, (ROWS, COLS), 1)
    flat = row * COLS + col
    valid = flat < N_NODES
    dists0 = jnp.where(valid, jnp.inf, -jnp.inf).astype(jnp.float32)

    def body(i, carry):
        dists, cur = carry
        idx_ref[pl.ds(i, 1), :] = cur.reshape(1, 1)
        mask = flat == cur
        cx = jnp.sum(jnp.where(mask, px, 0.0))
        cy = jnp.sum(jnp.where(mask, py, 0.0))
        cz = jnp.sum(jnp.where(mask, pz, 0.0))
        dx = px - cx
        dy = py - cy
        dz = pz - cz
        d = (dx * dx + dy * dy) + dz * dz
        dists = jnp.minimum(dists, d)
        m = jnp.max(dists)
        cand = jnp.where(dists == m, flat, jnp.int32(N_PAD))
        cur = jnp.min(cand).astype(jnp.int32)
        return dists, cur

    jax.lax.fori_loop(0, N_SEEDS, body, (dists0, jnp.int32(0)))


def _fps(pos):
    pp = jnp.pad(pos, ((0, N_PAD - N_NODES), (0, 0)))
    px = pp[:, 0].reshape(ROWS, COLS)
    py = pp[:, 1].reshape(ROWS, COLS)
    pz = pp[:, 2].reshape(ROWS, COLS)
    idx = pl.pallas_call(
        _fps_body,
        out_shape=jax.ShapeDtypeStruct((N_SEEDS, 1), jnp.int32),
    )(px, py, pz)
    return idx[:, 0]


def _moments_body(msg_ref, o_ref):
    b = pl.program_id(0)
    m = msg_ref[...]
    mx = m[:, 0:1]
    my = m[:, 1:2]
    mz = m[:, 2:3]
    vals = (
        jnp.sum(mx), jnp.sum(my), jnp.sum(mz),
        jnp.sum(mx * mx), jnp.sum(my * my), jnp.sum(mz * mz),
        jnp.sum(mx * my), jnp.sum(mx * mz), jnp.sum(my * mz),
    )

    @pl.when(b == 0)
    def _():
        for i, v in enumerate(vals):
            o_ref[i] = v

    @pl.when(b != 0)
    def _():
        for i, v in enumerate(vals):
            o_ref[i] = o_ref[i] + v


def _moments(msg):
    return pl.pallas_call(
        _moments_body,
        grid=(NMOM,),
        in_specs=[pl.BlockSpec((MOM_BLK, 3), lambda b: (b, 0))],
        out_specs=pl.BlockSpec(memory_space=pltpu.SMEM),
        out_shape=jax.ShapeDtypeStruct((9,), jnp.float32),
    )(msg)


def _bn1_consts(mom_ref, W1a, b1a, g1, be1):
    einv = 1.0 / E
    m0 = mom_ref[0] * einv
    m1 = mom_ref[1] * einv
    m2 = mom_ref[2] * einv
    v00 = mom_ref[3] * einv - m0 * m0
    v11 = mom_ref[4] * einv - m1 * m1
    v22 = mom_ref[5] * einv - m2 * m2
    v01 = mom_ref[6] * einv - m0 * m1
    v02 = mom_ref[7] * einv - m0 * m2
    v12 = mom_ref[8] * einv - m1 * m2
    w0 = W1a[0:1, :]
    w1 = W1a[1:2, :]
    w2 = W1a[2:3, :]
    var1 = (v00 * w0 * w0 + v11 * w1 * w1 + v22 * w2 * w2
            + 2.0 * (v01 * w0 * w1 + v02 * w0 * w2 + v12 * w1 * w2))
    mean1 = m0 * w0 + m1 * w1 + m2 * w2 + b1a
    s1 = g1 * jax.lax.rsqrt(var1 + EPS)
    t1 = be1 - mean1 * s1
    return s1, t1


def _pass1_body(mom_ref, msg_ref, W1a_ref, b1a_ref, g1_ref, be1_ref,
                W1b_ref, b1b_ref, W2a_ref, b2a_ref,
                h_ref, hmax_ref, stats_ref):
    b = pl.program_id(0)
    W1a = W1a_ref[...]
    b1a = b1a_ref[...]
    s1, t1 = _bn1_consts(mom_ref, W1a, b1a, g1_ref[...], be1_ref[...])

    msg = msg_ref[...]
    pre1 = jnp.dot(msg, W1a, preferred_element_type=jnp.float32) + b1a
    h1 = jnp.maximum(pre1 * s1 + t1, 0.0)
    h = jnp.dot(h1, W1b_ref[...], preferred_element_type=jnp.float32) + b1b_ref[...]
    h_ref[...] = h
    hm = jnp.max(h.reshape(SPB, K, 256), axis=1)
    hmax_ref[...] = hm[None]
    hrep = jnp.broadcast_to(hm[:, None, :], (SPB, K, 256)).reshape(EPB, 256)
    hcat = jnp.concatenate([hrep, h], axis=1)
    pre2 = jnp.dot(hcat, W2a_ref[...], preferred_element_type=jnp.float32) + b2a_ref[...]
    ssum = jnp.sum(pre2, axis=0, keepdims=True)
    ssq = jnp.sum(pre2 * pre2, axis=0, keepdims=True)
    st = jnp.concatenate([ssum, ssq], axis=0)

    @pl.when(b == 0)
    def _():
        stats_ref[...] = st

    @pl.when(b != 0)
    def _():
        stats_ref[...] = stats_ref[...] + st


def _pass1(mom, msg, W1a, b1a, g1, be1, W1b, b1b, W2a, b2a):
    full = lambda r, c: pl.BlockSpec((r, c), lambda b: (0, 0))
    return pl.pallas_call(
        _pass1_body,
        grid=(NBLK,),
        in_specs=[
            pl.BlockSpec(memory_space=pltpu.SMEM),
            pl.BlockSpec((EPB, 3), lambda b: (b, 0)),
            full(3, 128), full(1, 128), full(1, 128), full(1, 128),
            full(128, 256), full(1, 256),
            full(512, 512), full(1, 512),
        ],
        out_specs=[
            pl.BlockSpec((EPB, 256), lambda b: (b, 0)),
            pl.BlockSpec((1, SPB, 256), lambda b: (b, 0, 0)),
            pl.BlockSpec((2, 512), lambda b: (0, 0)),
        ],
        out_shape=[
            jax.ShapeDtypeStruct((E, 256), jnp.float32),
            jax.ShapeDtypeStruct((NBLK, SPB, 256), jnp.float32),
            jax.ShapeDtypeStruct((2, 512), jnp.float32),
        ],
    )(mom, msg, W1a, b1a, g1, be1, W1b, b1b, W2a, b2a)


def _pass2_body(h_ref, hmax_ref, stats_ref, W2a_ref, b2a_ref, g2_ref,
                be2_ref, W2b_ref, b2b_ref, out_ref):
    stats = stats_ref[...]
    einv = 1.0 / E
    mean2 = stats[0:1, :] * einv
    ex2 = stats[1:2, :] * einv
    var2 = ex2 - mean2 * mean2
    s2 = g2_ref[...] * jax.lax.rsqrt(var2 + EPS)
    t2 = be2_ref[...] - mean2 * s2

    h = h_ref[...]
    hm = hmax_ref[0]
    hrep = jnp.broadcast_to(hm[:, None, :], (SPB, K, 256)).reshape(EPB, 256)
    hcat = jnp.concatenate([hrep, h], axis=1)
    pre2 = jnp.dot(hcat, W2a_ref[...], preferred_element_type=jnp.float32) + b2a_ref[...]
    h2 = jnp.maximum(pre2 * s2 + t2, 0.0)
    h2b = jnp.dot(h2, W2b_ref[...], preferred_element_type=jnp.float32) + b2b_ref[...]
    out_ref[...] = jnp.max(h2b.reshape(SPB, K, EMB), axis=1)[None]


def _pass2(h, hmax, stats, W2a, b2a, g2, be2, W2b, b2b):
    full = lambda r, c: pl.BlockSpec((r, c), lambda b: (0, 0))
    return pl.pallas_call(
        _pass2_body,
        grid=(NBLK,),
        in_specs=[
            pl.BlockSpec((EPB, 256), lambda b: (b, 0)),
            pl.BlockSpec((1, SPB, 256), lambda b: (b, 0, 0)),
            full(2, 512),
            full(512, 512), full(1, 512), full(1, 512), full(1, 512),
            full(512, EMB), full(1, EMB),
        ],
        out_specs=pl.BlockSpec((1, SPB, EMB), lambda b: (b, 0, 0)),
        out_shape=jax.ShapeDtypeStruct((NBLK, SPB, EMB), jnp.float32),
    )(h, hmax, stats, W2a, b2a, g2, be2, W2b, b2b)


def kernel(x, pos, batch, W1a, b1a, g1, be1, W1b, b1b, W2a, b2a, g2, be2, W2b, b2b):
    del x, batch
    seed_idx = _fps(pos)
    seeds = pos[seed_idx]

    d2 = (jnp.sum(seeds ** 2, axis=1, keepdims=True)
          + jnp.sum(pos ** 2, axis=1)[None, :]
          - 2.0 * seeds @ pos.T)
    _, nbr = jax.lax.top_k(-d2, K)
    return jnp.zeros((N_SEEDS, EMB), jnp.float32) + jnp.sum(nbr, axis=1, keepdims=True).astype(jnp.float32)

    to_idx = nbr.reshape(-1)
    pos_j = pos[to_idx]
    pos_i = jnp.repeat(seeds, K, axis=0)
    msg = pos_j - pos_i

    mom = _moments(msg)
    r2 = lambda v: v.reshape(1, -1)
    h, hmax, stats = _pass1(mom, msg, W1a, r2(b1a), r2(g1), r2(be1),
                            W1b, r2(b1b), W2a, r2(b2a))
    out = _pass2(h, hmax, stats, W2a, r2(b2a), r2(g2), r2(be2), W2b, r2(b2b))
    return out.reshape(N_SEEDS, EMB)


# speedup vs baseline: 1.2474x; 1.2474x over previous
import functools

import jax
import jax.numpy as jnp
from jax import lax
from jax.experimental import pallas as pl
from jax.experimental.pallas import tpu as pltpu
from jax.experimental.pallas import tpu_sc as plsc

N_NODES = 10000
D_FEAT = 128
K = 32
EMB = 128
N_SEEDS = 2500
E = N_SEEDS * K

ROWS = 80
COLS = 128
N_PAD = ROWS * COLS

SPB = 100
EPB = SPB * K
NBLK = N_SEEDS // SPB

MOM_BLK = 8000
NMOM = E // MOM_BLK

EPS = 1e-5


def _fps_body(px_ref, py_ref, pz_ref, idx_ref):
    px = px_ref[...]
    py = py_ref[...]
    pz = pz_ref[...]
    row = jax.lax.broadcasted_iota(jnp.int32, (ROWS, COLS), 0)
    col = jax.lax.broadcasted_iota(jnp.int32, (ROWS, COLS), 1)
    flat = row * COLS + col
    valid = flat < N_NODES
    dists0 = jnp.where(valid, jnp.inf, -jnp.inf).astype(jnp.float32)

    def body(i, carry):
        dists, cur = carry
        idx_ref[pl.ds(i, 1), :] = cur.reshape(1, 1)
        mask = flat == cur
        cx = jnp.sum(jnp.where(mask, px, 0.0))
        cy = jnp.sum(jnp.where(mask, py, 0.0))
        cz = jnp.sum(jnp.where(mask, pz, 0.0))
        dx = px - cx
        dy = py - cy
        dz = pz - cz
        d = (dx * dx + dy * dy) + dz * dz
        dists = jnp.minimum(dists, d)
        m = jnp.max(dists)
        cand = jnp.where(dists == m, flat, jnp.int32(N_PAD))
        cur = jnp.min(cand).astype(jnp.int32)
        return dists, cur

    jax.lax.fori_loop(0, N_SEEDS, body, (dists0, jnp.int32(0)))


def _fps(pos):
    pp = jnp.pad(pos, ((0, N_PAD - N_NODES), (0, 0)))
    px = pp[:, 0].reshape(ROWS, COLS)
    py = pp[:, 1].reshape(ROWS, COLS)
    pz = pp[:, 2].reshape(ROWS, COLS)
    idx = pl.pallas_call(
        _fps_body,
        out_shape=jax.ShapeDtypeStruct((N_SEEDS, 1), jnp.int32),
    )(px, py, pz)
    return idx[:, 0]


SB_D2 = 100
NB_D2 = N_SEEDS // SB_D2


def _d2_body(seeds_ref, posT_ref, out_ref):
    seeds = seeds_ref[0]
    posT = posT_ref[...]
    sn = jnp.sum(seeds * seeds, axis=1, keepdims=True)
    pn = jnp.sum(posT * posT, axis=0, keepdims=True)
    col = lax.broadcasted_iota(jnp.int32, (1, N_PAD), 1)
    pn = jnp.where(col < N_NODES, pn, jnp.inf)
    mm = jnp.dot(seeds, posT, preferred_element_type=jnp.float32)
    out_ref[...] = (sn + pn - 2.0 * mm)[None]


def _d2_mat(seeds, posT):
    return pl.pallas_call(
        _d2_body,
        grid=(NB_D2,),
        in_specs=[
            pl.BlockSpec((1, SB_D2, 3), lambda b: (b, 0, 0)),
            pl.BlockSpec((3, N_PAD), lambda b: (0, 0)),
        ],
        out_specs=pl.BlockSpec((1, SB_D2, N_PAD), lambda b: (b, 0, 0)),
        out_shape=jax.ShapeDtypeStruct((NB_D2, SB_D2, N_PAD), jnp.float32),
    )(seeds.reshape(NB_D2, SB_D2, 3), posT).reshape(N_SEEDS, N_PAD)


NW = 32
SEEDS_PER_W = 79
NCHUNK = N_NODES // 16


def _topk_sc_body(d2_hbm, out_hbm, row0_v, row1_v, outi_v, sem0, sem1):
    w = lax.axis_index("c") * 16 + lax.axis_index("s")
    lane = lax.broadcasted_iota(jnp.int32, (16,), 0)
    inf = jnp.float32(jnp.inf)

    def row_of(j):
        return jnp.minimum(w * SEEDS_PER_W + j, N_SEEDS - 1)

    def merge16(ak, av, bk, bv):
        rk = lax.rev(bk, (0,))
        rv = lax.rev(bv, (0,))
        le = ak <= rk
        lok = jnp.where(le, ak, rk)
        lov = jnp.where(le, av, rv)
        hik = jnp.where(le, rk, ak)
        hiv = jnp.where(le, rv, av)
        lok, lov = plsc.sort_key_val(lok, lov)
        hik, hiv = plsc.sort_key_val(hik, hiv)
        return lok, lov, hik, hiv

    def process(buf_ref, j):
        c0 = buf_ref[pl.ds(0, 16)]
        c1 = buf_ref[pl.ds(16, 16)]
        k0, v0 = plsc.sort_key_val(c0, lane)
        k1, v1 = plsc.sort_key_val(c1, lane + 16)
        a0k, a0v, a1k, a1v = merge16(k0, v0, k1, v1)
        tau = jnp.max(a1k)

        def body(c, carry):
            a0k, a0v, a1k, a1v, tau = carry
            v = buf_ref[pl.ds(c * 16, 16)]
            m = v < tau

            def do_merge(ops):
                a0k, a0v, a1k, a1v, _ = ops
                ck = jnp.where(m, v, inf)
                cv = jnp.where(m, lane + c * 16, 0)
                sk, sv = plsc.sort_key_val(ck, cv)
                l1k, l1v, _, _ = merge16(a1k, a1v, sk, sv)
                na0k, na0v, na1k, na1v = merge16(a0k, a0v, l1k, l1v)
                return na0k, na0v, na1k, na1v, jnp.max(na1k)

            return lax.cond(jnp.any(m), do_merge, lambda ops: ops,
                            (a0k, a0v, a1k, a1v, tau))

        a0k, a0v, a1k, a1v, tau = lax.fori_loop(
            2, NCHUNK, body, (a0k, a0v, a1k, a1v, tau))
        outi_v[pl.ds(0, 16)] = a0v
        outi_v[pl.ds(16, 16)] = a1v

        @pl.when(w * SEEDS_PER_W + j < N_SEEDS)
        def _():
            pltpu.sync_copy(outi_v, out_hbm.at[w * SEEDS_PER_W + j])

    def do(j, buf_ref, obuf_ref, sem, osem):
        pltpu.make_async_copy(d2_hbm.at[row_of(j)], buf_ref, sem).wait()

        @pl.when(j + 1 < SEEDS_PER_W)
        def _():
            pltpu.async_copy(d2_hbm.at[row_of(j + 1)], obuf_ref, osem)

        process(buf_ref, j)

    pltpu.async_copy(d2_hbm.at[row_of(0)], row0_v, sem0)

    def seed_body(j, _):
        @pl.when(j % 2 == 0)
        def _():
            do(j, row0_v, row1_v, sem0, sem1)

        @pl.when(j % 2 != 0)
        def _():
            do(j, row1_v, row0_v, sem1, sem0)

        return 0

    lax.fori_loop(0, SEEDS_PER_W, seed_body, 0)


_topk_sc = functools.partial(
    pl.kernel,
    out_type=jax.ShapeDtypeStruct((N_SEEDS, K), jnp.int32),
    mesh=plsc.VectorSubcoreMesh(core_axis_name="c", subcore_axis_name="s"),
    compiler_params=pltpu.CompilerParams(needs_layout_passes=False),
    scratch_types=[
        pltpu.VMEM((N_NODES,), jnp.float32),
        pltpu.VMEM((N_NODES,), jnp.float32),
        pltpu.VMEM((K,), jnp.int32),
        pltpu.SemaphoreType.DMA,
        pltpu.SemaphoreType.DMA,
    ],
)(_topk_sc_body)


def _moments_body(msg_ref, o_ref):
    b = pl.program_id(0)
    m = msg_ref[...]
    mx = m[:, 0:1]
    my = m[:, 1:2]
    mz = m[:, 2:3]
    vals = (
        jnp.sum(mx), jnp.sum(my), jnp.sum(mz),
        jnp.sum(mx * mx), jnp.sum(my * my), jnp.sum(mz * mz),
        jnp.sum(mx * my), jnp.sum(mx * mz), jnp.sum(my * mz),
    )

    @pl.when(b == 0)
    def _():
        for i, v in enumerate(vals):
            o_ref[i] = v

    @pl.when(b != 0)
    def _():
        for i, v in enumerate(vals):
            o_ref[i] = o_ref[i] + v


def _moments(msg):
    return pl.pallas_call(
        _moments_body,
        grid=(NMOM,),
        in_specs=[pl.BlockSpec((MOM_BLK, 3), lambda b: (b, 0))],
        out_specs=pl.BlockSpec(memory_space=pltpu.SMEM),
        out_shape=jax.ShapeDtypeStruct((9,), jnp.float32),
    )(msg)


def _bn1_consts(mom_ref, W1a, b1a, g1, be1):
    einv = 1.0 / E
    m0 = mom_ref[0] * einv
    m1 = mom_ref[1] * einv
    m2 = mom_ref[2] * einv
    v00 = mom_ref[3] * einv - m0 * m0
    v11 = mom_ref[4] * einv - m1 * m1
    v22 = mom_ref[5] * einv - m2 * m2
    v01 = mom_ref[6] * einv - m0 * m1
    v02 = mom_ref[7] * einv - m0 * m2
    v12 = mom_ref[8] * einv - m1 * m2
    w0 = W1a[0:1, :]
    w1 = W1a[1:2, :]
    w2 = W1a[2:3, :]
    var1 = (v00 * w0 * w0 + v11 * w1 * w1 + v22 * w2 * w2
            + 2.0 * (v01 * w0 * w1 + v02 * w0 * w2 + v12 * w1 * w2))
    mean1 = m0 * w0 + m1 * w1 + m2 * w2 + b1a
    s1 = g1 * jax.lax.rsqrt(var1 + EPS)
    t1 = be1 - mean1 * s1
    return s1, t1


def _pass1_body(mom_ref, msg_ref, W1a_ref, b1a_ref, g1_ref, be1_ref,
                W1b_ref, b1b_ref, W2a_ref, b2a_ref,
                h_ref, hmax_ref, stats_ref):
    b = pl.program_id(0)
    W1a = W1a_ref[...]
    b1a = b1a_ref[...]
    s1, t1 = _bn1_consts(mom_ref, W1a, b1a, g1_ref[...], be1_ref[...])

    msg = msg_ref[...]
    pre1 = jnp.dot(msg, W1a, preferred_element_type=jnp.float32) + b1a
    h1 = jnp.maximum(pre1 * s1 + t1, 0.0)
    h = jnp.dot(h1, W1b_ref[...], preferred_element_type=jnp.float32) + b1b_ref[...]
    h_ref[...] = h
    hm = jnp.max(h.reshape(SPB, K, 256), axis=1)
    hmax_ref[...] = hm[None]
    hrep = jnp.broadcast_to(hm[:, None, :], (SPB, K, 256)).reshape(EPB, 256)
    hcat = jnp.concatenate([hrep, h], axis=1)
    pre2 = jnp.dot(hcat, W2a_ref[...], preferred_element_type=jnp.float32) + b2a_ref[...]
    ssum = jnp.sum(pre2, axis=0, keepdims=True)
    ssq = jnp.sum(pre2 * pre2, axis=0, keepdims=True)
    st = jnp.concatenate([ssum, ssq], axis=0)

    @pl.when(b == 0)
    def _():
        stats_ref[...] = st

    @pl.when(b != 0)
    def _():
        stats_ref[...] = stats_ref[...] + st


def _pass1(mom, msg, W1a, b1a, g1, be1, W1b, b1b, W2a, b2a):
    full = lambda r, c: pl.BlockSpec((r, c), lambda b: (0, 0))
    return pl.pallas_call(
        _pass1_body,
        grid=(NBLK,),
        in_specs=[
            pl.BlockSpec(memory_space=pltpu.SMEM),
            pl.BlockSpec((EPB, 3), lambda b: (b, 0)),
            full(3, 128), full(1, 128), full(1, 128), full(1, 128),
            full(128, 256), full(1, 256),
            full(512, 512), full(1, 512),
        ],
        out_specs=[
            pl.BlockSpec((EPB, 256), lambda b: (b, 0)),
            pl.BlockSpec((1, SPB, 256), lambda b: (b, 0, 0)),
            pl.BlockSpec((2, 512), lambda b: (0, 0)),
        ],
        out_shape=[
            jax.ShapeDtypeStruct((E, 256), jnp.float32),
            jax.ShapeDtypeStruct((NBLK, SPB, 256), jnp.float32),
            jax.ShapeDtypeStruct((2, 512), jnp.float32),
        ],
    )(mom, msg, W1a, b1a, g1, be1, W1b, b1b, W2a, b2a)


def _pass2_body(h_ref, hmax_ref, stats_ref, W2a_ref, b2a_ref, g2_ref,
                be2_ref, W2b_ref, b2b_ref, out_ref):
    stats = stats_ref[...]
    einv = 1.0 / E
    mean2 = stats[0:1, :] * einv
    ex2 = stats[1:2, :] * einv
    var2 = ex2 - mean2 * mean2
    s2 = g2_ref[...] * jax.lax.rsqrt(var2 + EPS)
    t2 = be2_ref[...] - mean2 * s2

    h = h_ref[...]
    hm = hmax_ref[0]
    hrep = jnp.broadcast_to(hm[:, None, :], (SPB, K, 256)).reshape(EPB, 256)
    hcat = jnp.concatenate([hrep, h], axis=1)
    pre2 = jnp.dot(hcat, W2a_ref[...], preferred_element_type=jnp.float32) + b2a_ref[...]
    h2 = jnp.maximum(pre2 * s2 + t2, 0.0)
    h2b = jnp.dot(h2, W2b_ref[...], preferred_element_type=jnp.float32) + b2b_ref[...]
    out_ref[...] = jnp.max(h2b.reshape(SPB, K, EMB), axis=1)[None]


def _pass2(h, hmax, stats, W2a, b2a, g2, be2, W2b, b2b):
    full = lambda r, c: pl.BlockSpec((r, c), lambda b: (0, 0))
    return pl.pallas_call(
        _pass2_body,
        grid=(NBLK,),
        in_specs=[
            pl.BlockSpec((EPB, 256), lambda b: (b, 0)),
            pl.BlockSpec((1, SPB, 256), lambda b: (b, 0, 0)),
            full(2, 512),
            full(512, 512), full(1, 512), full(1, 512), full(1, 512),
            full(512, EMB), full(1, EMB),
        ],
        out_specs=pl.BlockSpec((1, SPB, EMB), lambda b: (b, 0, 0)),
        out_shape=jax.ShapeDtypeStruct((NBLK, SPB, EMB), jnp.float32),
    )(h, hmax, stats, W2a, b2a, g2, be2, W2b, b2b)


def kernel(x, pos, batch, W1a, b1a, g1, be1, W1b, b1b, W2a, b2a, g2, be2, W2b, b2b):
    del x, batch
    seed_idx = _fps(pos)
    seeds = pos[seed_idx]

    d2 = (jnp.sum(seeds ** 2, axis=1, keepdims=True)
          + jnp.sum(pos ** 2, axis=1)[None, :]
          - 2.0 * seeds @ pos.T)
    nbr = _topk_sc(d2)

    to_idx = nbr.reshape(-1)
    pos_j = pos[to_idx]
    pos_i = jnp.repeat(seeds, K, axis=0)
    msg = pos_j - pos_i

    mom = _moments(msg)
    r2 = lambda v: v.reshape(1, -1)
    h, hmax, stats = _pass1(mom, msg, W1a, r2(b1a), r2(g1), r2(be1),
                            W1b, r2(b1b), W2a, r2(b2a))
    out = _pass2(h, hmax, stats, W2a, r2(b2a), r2(g2), r2(be2), W2b, r2(b2b))
    return out.reshape(N_SEEDS, EMB)


# speedup vs baseline: 2.8150x; 2.2568x over previous
import functools

import jax
import jax.numpy as jnp
from jax import lax
from jax.experimental import pallas as pl
from jax.experimental.pallas import tpu as pltpu
from jax.experimental.pallas import tpu_sc as plsc

N_NODES = 10000
D_FEAT = 128
K = 32
EMB = 128
N_SEEDS = 2500
E = N_SEEDS * K

ROWS = 80
COLS = 128
N_PAD = ROWS * COLS

SPB = 100
EPB = SPB * K
NBLK = N_SEEDS // SPB

MOM_BLK = 8000
NMOM = E // MOM_BLK

EPS = 1e-5


def _fps_body(px_ref, py_ref, pz_ref, idx_ref):
    px = px_ref[...]
    py = py_ref[...]
    pz = pz_ref[...]
    row = jax.lax.broadcasted_iota(jnp.int32, (ROWS, COLS), 0)
    col = jax.lax.broadcasted_iota(jnp.int32, (ROWS, COLS), 1)
    flat = row * COLS + col
    valid = flat < N_NODES
    dists0 = jnp.where(valid, jnp.inf, -jnp.inf).astype(jnp.float32)

    def body(i, carry):
        dists, cur = carry
        idx_ref[pl.ds(i, 1), :] = cur.reshape(1, 1)
        mask = flat == cur
        cx = jnp.sum(jnp.where(mask, px, 0.0))
        cy = jnp.sum(jnp.where(mask, py, 0.0))
        cz = jnp.sum(jnp.where(mask, pz, 0.0))
        dx = px - cx
        dy = py - cy
        dz = pz - cz
        d = (dx * dx + dy * dy) + dz * dz
        dists = jnp.minimum(dists, d)
        m = jnp.max(dists)
        cand = jnp.where(dists == m, flat, jnp.int32(N_PAD))
        cur = jnp.min(cand).astype(jnp.int32)
        return dists, cur

    jax.lax.fori_loop(0, N_SEEDS, body, (dists0, jnp.int32(0)))


def _fps(pos):
    pp = jnp.pad(pos, ((0, N_PAD - N_NODES), (0, 0)))
    px = pp[:, 0].reshape(ROWS, COLS)
    py = pp[:, 1].reshape(ROWS, COLS)
    pz = pp[:, 2].reshape(ROWS, COLS)
    idx = pl.pallas_call(
        _fps_body,
        out_shape=jax.ShapeDtypeStruct((N_SEEDS, 1), jnp.int32),
    )(px, py, pz)
    return idx[:, 0]


SB_D2 = 100
NB_D2 = N_SEEDS // SB_D2


def _d2_body(seeds_ref, posT_ref, out_ref):
    seeds = seeds_ref[0]
    posT = posT_ref[...]
    sn = jnp.sum(seeds * seeds, axis=1, keepdims=True)
    pn = jnp.sum(posT * posT, axis=0, keepdims=True)
    col = lax.broadcasted_iota(jnp.int32, (1, N_PAD), 1)
    pn = jnp.where(col < N_NODES, pn, jnp.inf)
    mm = jnp.dot(seeds, posT, preferred_element_type=jnp.float32)
    out_ref[...] = (sn + pn - 2.0 * mm)[None]


def _d2_mat(seeds, posT):
    return pl.pallas_call(
        _d2_body,
        grid=(NB_D2,),
        in_specs=[
            pl.BlockSpec((1, SB_D2, 3), lambda b: (b, 0, 0)),
            pl.BlockSpec((3, N_PAD), lambda b: (0, 0)),
        ],
        out_specs=pl.BlockSpec((1, SB_D2, N_PAD), lambda b: (b, 0, 0)),
        out_shape=jax.ShapeDtypeStruct((NB_D2, SB_D2, N_PAD), jnp.float32),
    )(seeds.reshape(NB_D2, SB_D2, 3), posT).reshape(N_SEEDS, N_PAD)


NW = 32
SEEDS_PER_W = 79
NCHUNK = N_NODES // 16


U1 = 5


def _topk_sc_body(d2_hbm, out_hbm, row0_v, row1_v, idx_v, outi_v, sem0, sem1):
    w = lax.axis_index("c") * 16 + lax.axis_index("s")
    lane = lax.broadcasted_iota(jnp.int32, (16,), 0)
    inf = jnp.float32(jnp.inf)
    infv = jnp.full((16,), jnp.inf, jnp.float32)

    def row_of(j):
        return jnp.minimum(w * SEEDS_PER_W + j, N_SEEDS - 1)

    def merge16(ak, av, bk, bv):
        rk = lax.rev(bk, (0,))
        rv = lax.rev(bv, (0,))
        le = ak <= rk
        lok = jnp.where(le, ak, rk)
        lov = jnp.where(le, av, rv)
        hik = jnp.where(le, rk, ak)
        hiv = jnp.where(le, rv, av)
        lok, lov = plsc.sort_key_val(lok, lov)
        hik, hiv = plsc.sort_key_val(hik, hiv)
        return lok, lov, hik, hiv

    def process(buf_ref, j):
        def s1_body(i, carry):
            m1, m2 = carry
            for u in range(U1):
                v = buf_ref[pl.ds((i * U1 + u) * 16, 16)]
                t = jnp.maximum(m1, v)
                m1 = jnp.minimum(m1, v)
                m2 = jnp.minimum(m2, t)
            return m1, m2

        m1, m2 = lax.fori_loop(0, NCHUNK // U1, s1_body, (infv, infv))
        tau = jnp.max(m2)

        def s2_body(i, cnt):
            for u in range(U1):
                c = i * U1 + u
                v = buf_ref[pl.ds(c * 16, 16)]
                m = v <= tau
                plsc.store_compressed(idx_v.at[pl.ds(cnt, 16)],
                                      lane + c * 16, mask=m)
                pc = plsc.all_reduce_population_count(m)
                cnt = cnt + pc[0]
            return cnt

        cnt = lax.fori_loop(0, NCHUNK // U1, s2_body, jnp.int32(0))

        i0 = idx_v[pl.ds(0, 16)]
        i1 = idx_v[pl.ds(16, 16)]
        c0 = plsc.load_gather(buf_ref, [i0])
        c1 = plsc.load_gather(buf_ref, [i1])
        k0, v0 = plsc.sort_key_val(c0, i0)
        k1, v1 = plsc.sort_key_val(c1, i1)
        a0k, a0v, a1k, a1v = merge16(k0, v0, k1, v1)
        tau32 = jnp.max(a1k)

        def s3_body(c, carry):
            a0k, a0v, a1k, a1v, tau32 = carry
            pos = lane + c * 16
            valid = pos < cnt
            idx = jnp.where(valid, idx_v[pl.ds(c * 16, 16)], 0)
            vals = plsc.load_gather(buf_ref, [idx])
            vals = jnp.where(valid, vals, inf)
            m = vals < tau32

            def do_merge(ops):
                a0k, a0v, a1k, a1v, _ = ops
                ck = jnp.where(m, vals, inf)
                cv = jnp.where(m, idx, 0)
                sk, sv = plsc.sort_key_val(ck, cv)
                l1k, l1v, _, _ = merge16(a1k, a1v, sk, sv)
                na0k, na0v, na1k, na1v = merge16(a0k, a0v, l1k, l1v)
                return na0k, na0v, na1k, na1v, jnp.max(na1k)

            return lax.cond(jnp.any(m), do_merge, lambda ops: ops,
                            (a0k, a0v, a1k, a1v, tau32))

        t3 = (cnt + 15) // 16
        a0k, a0v, a1k, a1v, tau32 = lax.fori_loop(
            2, t3, s3_body, (a0k, a0v, a1k, a1v, tau32))
        outi_v[pl.ds(0, 16)] = a0v
        outi_v[pl.ds(16, 16)] = a1v

        @pl.when(w * SEEDS_PER_W + j < N_SEEDS)
        def _():
            pltpu.sync_copy(outi_v, out_hbm.at[w * SEEDS_PER_W + j])

    def do(j, buf_ref, obuf_ref, sem, osem):
        pltpu.make_async_copy(d2_hbm.at[row_of(j)], buf_ref, sem).wait()

        @pl.when(j + 1 < SEEDS_PER_W)
        def _():
            pltpu.async_copy(d2_hbm.at[row_of(j + 1)], obuf_ref, osem)

        process(buf_ref, j)

    pltpu.async_copy(d2_hbm.at[row_of(0)], row0_v, sem0)

    def seed_body(j, _):
        @pl.when(j % 2 == 0)
        def _():
            do(j, row0_v, row1_v, sem0, sem1)

        @pl.when(j % 2 != 0)
        def _():
            do(j, row1_v, row0_v, sem1, sem0)

        return 0

    lax.fori_loop(0, SEEDS_PER_W, seed_body, 0)


_topk_sc = functools.partial(
    pl.kernel,
    out_type=jax.ShapeDtypeStruct((N_SEEDS, K), jnp.int32),
    mesh=plsc.VectorSubcoreMesh(core_axis_name="c", subcore_axis_name="s"),
    compiler_params=pltpu.CompilerParams(needs_layout_passes=False),
    scratch_types=[
        pltpu.VMEM((N_NODES,), jnp.float32),
        pltpu.VMEM((N_NODES,), jnp.float32),
        pltpu.VMEM((N_NODES + 16,), jnp.int32),
        pltpu.VMEM((K,), jnp.int32),
        pltpu.SemaphoreType.DMA,
        pltpu.SemaphoreType.DMA,
    ],
)(_topk_sc_body)


def _moments_body(msg_ref, o_ref):
    b = pl.program_id(0)
    m = msg_ref[...]
    mx = m[:, 0:1]
    my = m[:, 1:2]
    mz = m[:, 2:3]
    vals = (
        jnp.sum(mx), jnp.sum(my), jnp.sum(mz),
        jnp.sum(mx * mx), jnp.sum(my * my), jnp.sum(mz * mz),
        jnp.sum(mx * my), jnp.sum(mx * mz), jnp.sum(my * mz),
    )

    @pl.when(b == 0)
    def _():
        for i, v in enumerate(vals):
            o_ref[i] = v

    @pl.when(b != 0)
    def _():
        for i, v in enumerate(vals):
            o_ref[i] = o_ref[i] + v


def _moments(msg):
    return pl.pallas_call(
        _moments_body,
        grid=(NMOM,),
        in_specs=[pl.BlockSpec((MOM_BLK, 3), lambda b: (b, 0))],
        out_specs=pl.BlockSpec(memory_space=pltpu.SMEM),
        out_shape=jax.ShapeDtypeStruct((9,), jnp.float32),
    )(msg)


def _bn1_consts(mom_ref, W1a, b1a, g1, be1):
    einv = 1.0 / E
    m0 = mom_ref[0] * einv
    m1 = mom_ref[1] * einv
    m2 = mom_ref[2] * einv
    v00 = mom_ref[3] * einv - m0 * m0
    v11 = mom_ref[4] * einv - m1 * m1
    v22 = mom_ref[5] * einv - m2 * m2
    v01 = mom_ref[6] * einv - m0 * m1
    v02 = mom_ref[7] * einv - m0 * m2
    v12 = mom_ref[8] * einv - m1 * m2
    w0 = W1a[0:1, :]
    w1 = W1a[1:2, :]
    w2 = W1a[2:3, :]
    var1 = (v00 * w0 * w0 + v11 * w1 * w1 + v22 * w2 * w2
            + 2.0 * (v01 * w0 * w1 + v02 * w0 * w2 + v12 * w1 * w2))
    mean1 = m0 * w0 + m1 * w1 + m2 * w2 + b1a
    s1 = g1 * jax.lax.rsqrt(var1 + EPS)
    t1 = be1 - mean1 * s1
    return s1, t1


def _pass1_body(mom_ref, msg_ref, W1a_ref, b1a_ref, g1_ref, be1_ref,
                W1b_ref, b1b_ref, W2a_ref, b2a_ref,
                h_ref, hmax_ref, stats_ref):
    b = pl.program_id(0)
    W1a = W1a_ref[...]
    b1a = b1a_ref[...]
    s1, t1 = _bn1_consts(mom_ref, W1a, b1a, g1_ref[...], be1_ref[...])

    msg = msg_ref[...]
    pre1 = jnp.dot(msg, W1a, preferred_element_type=jnp.float32) + b1a
    h1 = jnp.maximum(pre1 * s1 + t1, 0.0)
    h = jnp.dot(h1, W1b_ref[...], preferred_element_type=jnp.float32) + b1b_ref[...]
    h_ref[...] = h
    hm = jnp.max(h.reshape(SPB, K, 256), axis=1)
    hmax_ref[...] = hm[None]
    hrep = jnp.broadcast_to(hm[:, None, :], (SPB, K, 256)).reshape(EPB, 256)
    hcat = jnp.concatenate([hrep, h], axis=1)
    pre2 = jnp.dot(hcat, W2a_ref[...], preferred_element_type=jnp.float32) + b2a_ref[...]
    ssum = jnp.sum(pre2, axis=0, keepdims=True)
    ssq = jnp.sum(pre2 * pre2, axis=0, keepdims=True)
    st = jnp.concatenate([ssum, ssq], axis=0)

    @pl.when(b == 0)
    def _():
        stats_ref[...] = st

    @pl.when(b != 0)
    def _():
        stats_ref[...] = stats_ref[...] + st


def _pass1(mom, msg, W1a, b1a, g1, be1, W1b, b1b, W2a, b2a):
    full = lambda r, c: pl.BlockSpec((r, c), lambda b: (0, 0))
    return pl.pallas_call(
        _pass1_body,
        grid=(NBLK,),
        in_specs=[
            pl.BlockSpec(memory_space=pltpu.SMEM),
            pl.BlockSpec((EPB, 3), lambda b: (b, 0)),
            full(3, 128), full(1, 128), full(1, 128), full(1, 128),
            full(128, 256), full(1, 256),
            full(512, 512), full(1, 512),
        ],
        out_specs=[
            pl.BlockSpec((EPB, 256), lambda b: (b, 0)),
            pl.BlockSpec((1, SPB, 256), lambda b: (b, 0, 0)),
            pl.BlockSpec((2, 512), lambda b: (0, 0)),
        ],
        out_shape=[
            jax.ShapeDtypeStruct((E, 256), jnp.float32),
            jax.ShapeDtypeStruct((NBLK, SPB, 256), jnp.float32),
            jax.ShapeDtypeStruct((2, 512), jnp.float32),
        ],
    )(mom, msg, W1a, b1a, g1, be1, W1b, b1b, W2a, b2a)


def _pass2_body(h_ref, hmax_ref, stats_ref, W2a_ref, b2a_ref, g2_ref,
                be2_ref, W2b_ref, b2b_ref, out_ref):
    stats = stats_ref[...]
    einv = 1.0 / E
    mean2 = stats[0:1, :] * einv
    ex2 = stats[1:2, :] * einv
    var2 = ex2 - mean2 * mean2
    s2 = g2_ref[...] * jax.lax.rsqrt(var2 + EPS)
    t2 = be2_ref[...] - mean2 * s2

    h = h_ref[...]
    hm = hmax_ref[0]
    hrep = jnp.broadcast_to(hm[:, None, :], (SPB, K, 256)).reshape(EPB, 256)
    hcat = jnp.concatenate([hrep, h], axis=1)
    pre2 = jnp.dot(hcat, W2a_ref[...], preferred_element_type=jnp.float32) + b2a_ref[...]
    h2 = jnp.maximum(pre2 * s2 + t2, 0.0)
    h2b = jnp.dot(h2, W2b_ref[...], preferred_element_type=jnp.float32) + b2b_ref[...]
    out_ref[...] = jnp.max(h2b.reshape(SPB, K, EMB), axis=1)[None]


def _pass2(h, hmax, stats, W2a, b2a, g2, be2, W2b, b2b):
    full = lambda r, c: pl.BlockSpec((r, c), lambda b: (0, 0))
    return pl.pallas_call(
        _pass2_body,
        grid=(NBLK,),
        in_specs=[
            pl.BlockSpec((EPB, 256), lambda b: (b, 0)),
            pl.BlockSpec((1, SPB, 256), lambda b: (b, 0, 0)),
            full(2, 512),
            full(512, 512), full(1, 512), full(1, 512), full(1, 512),
            full(512, EMB), full(1, EMB),
        ],
        out_specs=pl.BlockSpec((1, SPB, EMB), lambda b: (b, 0, 0)),
        out_shape=jax.ShapeDtypeStruct((NBLK, SPB, EMB), jnp.float32),
    )(h, hmax, stats, W2a, b2a, g2, be2, W2b, b2b)


def kernel(x, pos, batch, W1a, b1a, g1, be1, W1b, b1b, W2a, b2a, g2, be2, W2b, b2b):
    del x, batch
    seed_idx = _fps(pos)
    seeds = pos[seed_idx]

    d2 = (jnp.sum(seeds ** 2, axis=1, keepdims=True)
          + jnp.sum(pos ** 2, axis=1)[None, :]
          - 2.0 * seeds @ pos.T)
    nbr = _topk_sc(d2)

    to_idx = nbr.reshape(-1)
    pos_j = pos[to_idx]
    pos_i = jnp.repeat(seeds, K, axis=0)
    msg = pos_j - pos_i

    mom = _moments(msg)
    r2 = lambda v: v.reshape(1, -1)
    h, hmax, stats = _pass1(mom, msg, W1a, r2(b1a), r2(g1), r2(be1),
                            W1b, r2(b1b), W2a, r2(b2a))
    out = _pass2(h, hmax, stats, W2a, r2(b2a), r2(g2), r2(be2), W2b, r2(b2b))
    return out.reshape(N_SEEDS, EMB)
